# baseline (speedup 1.0000x reference)
import functools

import jax
import jax.numpy as jnp
from jax import lax
from jax.experimental import pallas as pl
from jax.experimental.pallas import tpu as pltpu
from jax.experimental.pallas import tpu_sc as plsc

NC, NS, L = 2, 16, 16
NW = NC * NS
B = 16384
D = 64
N_ROWS = 1000000
FULL_TILES = N_ROWS // 128
TAIL_LO = FULL_TILES * 128
CHUNK = 256
XROWS = B + 8
SEL_CAP = 4096
DUMP = B

_mesh = plsc.VectorSubcoreMesh(core_axis_name="c", subcore_axis_name="s")


@functools.partial(
    pl.kernel,
    out_type=(
        jax.ShapeDtypeStruct((XROWS, 128), jnp.float32),
        jax.ShapeDtypeStruct((XROWS, 128), jnp.float32),
    ),
    mesh=_mesh,
    scratch_types=[
        pltpu.VMEM((B,), jnp.int32),
        pltpu.VMEM((B,), jnp.int32),
        pltpu.VMEM((SEL_CAP,), jnp.int32),
        pltpu.VMEM((3, D, CHUNK), jnp.float32),
        pltpu.VMEM((2, 128, 128), jnp.float32),
        pltpu.VMEM((2, 128), jnp.int32),
        pltpu.SemaphoreType.DMA,
        pltpu.SemaphoreType.DMA,
        pltpu.SemaphoreType.DMA,
    ],
    compiler_params=pltpu.CompilerParams(
        needs_layout_passes=False, use_tc_tiling_on_sc=True
    ),
)
def _mf_gather(user_hbm, item_hbm, uwt_hbm, iwt_hbm, tailu_hbm, taili_hbm,
               xu_hbm, xi_hbm,
               idx_all, pb, selb, stage3, rowst3, sidx2,
               sem_st, sem_s0, sem_s1):
    wid = lax.axis_index("s") * NC + lax.axis_index("c")
    lane = lax.iota(jnp.int32, L)
    count_t = jnp.where(wid >= 28, 245, 244)
    start_t = 244 * wid + jnp.maximum(wid - 28, 0)
    r_slab_lo = start_t * 128
    r_slab_hi = jnp.where(wid == NW - 1, N_ROWS, (start_t + count_t) * 128)
    nchunk = (count_t + 1) >> 1
    nchunk_eff = nchunk + jnp.where(wid == NW - 1, 1, 0)

    def sweep(idx_hbm, tbl_hbm, tail_hbm, x_hbm):
        pltpu.sync_copy(idx_hbm, idx_all)

        def scan_all(q, n):
            bvec = q * L + lane
            rvec = idx_all[pl.ds(q * L, L)]
            m = (rvec >= r_slab_lo) & (rvec < r_slab_hi)
            c = plsc.cumsum(m.astype(jnp.int32))
            plsc.store_scatter(pb, [n + c - 1], bvec, mask=m)
            return n + c[L - 1]

        n_pairs = lax.fori_loop(0, B // L, scan_all, 0)
        nq = (n_pairs + 15) >> 4

        def start_chunk_dma(ci):
            off = (start_t + jnp.minimum(2 * ci, count_t - 2)) * 128
            pltpu.async_copy(
                tbl_hbm.at[:, pl.ds(off, CHUNK)], stage3.at[ci % 3], sem_st
            )

        start_chunk_dma(0)

        def chunk_body(ci, par, sem_sc):
            is_tail = ci == nchunk
            sl = ci % 3
            r_lo = jnp.where(
                is_tail, TAIL_LO,
                (start_t + jnp.minimum(2 * ci, count_t - 2)) * 128,
            )

            @pl.when(ci + 1 < nchunk)
            def _():
                start_chunk_dma(ci + 1)

            @pl.when((ci + 1 == nchunk) & (wid == NW - 1))
            def _():
                pltpu.async_copy(
                    tail_hbm, stage3.at[(ci + 1) % 3, :, pl.ds(0, 128)], sem_st
                )

            @pl.when(jnp.logical_not(is_tail))
            def _():
                pltpu.make_async_copy(
                    tbl_hbm.at[:, pl.ds(0, CHUNK)], stage3.at[sl], sem_st
                ).wait()

            @pl.when(is_tail)
            def _():
                pltpu.make_async_copy(
                    tail_hbm, stage3.at[sl, :, pl.ds(0, 128)], sem_st
                ).wait()

            @pl.when(ci >= 2)
            def _():
                pltpu.make_async_copy(
                    rowst3.at[par], x_hbm.at[plsc.Indices(sidx2.at[par], ignored_value=DUMP)], sem_sc
                ).wait()

            for k in range(8):
                sidx2[par, pl.ds(k * L, L)] = jnp.full((L,), DUMP, jnp.int32)

            def scan_pairs(q, n):
                valid = (q * L + lane) < n_pairs
                bvec = pb[pl.ds(q * L, L)]
                bs = jnp.minimum(jnp.maximum(bvec, 0), B - 1)
                rvec = plsc.load_gather(idx_all, [bs])
                m = valid & (rvec >= r_lo) & (rvec < r_lo + CHUNK)
                c = plsc.cumsum(m.astype(jnp.int32))
                pos = jnp.minimum(n + c - 1, SEL_CAP - 1)
                plsc.store_scatter(selb, [pos], bvec, mask=m)
                return n + c[L - 1]

            n_sel = lax.fori_loop(0, nq, scan_pairs, 0)
            ngroups = (n_sel + 15) >> 4

            par_v = jnp.full((L,), par, jnp.int32)
            sl_v = jnp.broadcast_to(sl, (L,)).astype(jnp.int32)

            def group(g, carry):
                valid = (g * L + lane) < n_sel
                b16 = selb[pl.ds((g % (SEL_CAP // L)) * L, L)]
                b16 = jnp.where(valid, b16, DUMP)
                bs = jnp.minimum(jnp.maximum(b16, 0), B - 1)
                rloc = plsc.load_gather(idx_all, [bs]) - r_lo
                rloc = jnp.minimum(jnp.maximum(rloc, 0), CHUNK - 1)
                slot = g % 8
                row16 = slot * L + lane
                for d in range(D):
                    d_v = jnp.full((L,), d, jnp.int32)
                    vals = plsc.load_gather(stage3, [sl_v, d_v, rloc])
                    plsc.store_scatter(rowst3, [par_v, row16, d_v], vals)
                sidx2[par, pl.ds(slot * L, L)] = b16

                @pl.when((slot == 7) & (g + 1 < ngroups))
                def _():
                    pltpu.async_copy(
                        rowst3.at[par], x_hbm.at[plsc.Indices(sidx2.at[par], ignored_value=DUMP)], sem_sc
                    ).wait()

                return carry

            lax.fori_loop(0, ngroups, group, 0)
            pltpu.async_copy(rowst3.at[par], x_hbm.at[plsc.Indices(sidx2.at[par], ignored_value=DUMP)], sem_sc)

        def body(ci, carry):
            @pl.when(ci % 2 == 0)
            def _():
                chunk_body(ci, 0, sem_s0)

            @pl.when(ci % 2 == 1)
            def _():
                chunk_body(ci, 1, sem_s1)

            return carry

        lax.fori_loop(0, nchunk_eff, body, 0)

        pltpu.make_async_copy(
            rowst3.at[0], x_hbm.at[plsc.Indices(sidx2.at[0], ignored_value=DUMP)], sem_s0
        ).wait()
        pltpu.make_async_copy(
            rowst3.at[1], x_hbm.at[plsc.Indices(sidx2.at[1], ignored_value=DUMP)], sem_s1
        ).wait()

    sweep(user_hbm, uwt_hbm, tailu_hbm, xu_hbm)
    sweep(item_hbm, iwt_hbm, taili_hbm, xi_hbm)


def _dot_body(xu_ref, xi_ref, o_ref):
    u = xu_ref[:, :D]
    it = xi_ref[:, :D]
    o_ref[...] = jnp.sum(u * it, axis=1)


_dot = pl.pallas_call(
    _dot_body,
    grid=(16,),
    in_specs=[
        pl.BlockSpec((1024, 128), lambda i: (i, 0)),
        pl.BlockSpec((1024, 128), lambda i: (i, 0)),
    ],
    out_specs=pl.BlockSpec((1024,), lambda i: (i,)),
    out_shape=jax.ShapeDtypeStruct((B,), jnp.float32),
)


def kernel(user, item, user_weight, item_weight):
    uwt = user_weight.T
    iwt = item_weight.T
    tail_u = jnp.pad(uwt[:, TAIL_LO:], ((0, 0), (0, 64)))
    tail_i = jnp.pad(iwt[:, TAIL_LO:], ((0, 0), (0, 64)))
    xu, xi = _mf_gather(user, item, uwt, iwt, tail_u, tail_i)
    return _dot(xu[:B], xi[:B])


# speedup vs baseline: 1.4596x; 1.4596x over previous
import functools

import jax
import jax.numpy as jnp
from jax import lax
from jax.experimental import pallas as pl
from jax.experimental.pallas import tpu as pltpu
from jax.experimental.pallas import tpu_sc as plsc

NC, NS, L = 2, 16, 16
NW = NC * NS
B = 16384
D = 64
N_ROWS = 1000000
FULL_TILES = N_ROWS // 128
TAIL_LO = FULL_TILES * 128
CHUNK = 512
XROWS = B + 8
SEL_CAP = 2048
DUMP = B

_mesh = plsc.VectorSubcoreMesh(core_axis_name="c", subcore_axis_name="s")


@functools.partial(
    pl.kernel,
    out_type=(
        jax.ShapeDtypeStruct((XROWS, 128), jnp.float32),
        jax.ShapeDtypeStruct((XROWS, 128), jnp.float32),
    ),
    mesh=_mesh,
    scratch_types=[
        pltpu.VMEM((B,), jnp.int32),
        pltpu.VMEM((B,), jnp.int32),
        pltpu.VMEM((SEL_CAP,), jnp.int32),
        pltpu.VMEM((2, D, CHUNK), jnp.float32),
        pltpu.VMEM((2, 64, 128), jnp.float32),
        pltpu.VMEM((2, 64), jnp.int32),
        pltpu.SemaphoreType.DMA,
        pltpu.SemaphoreType.DMA,
        pltpu.SemaphoreType.DMA,
    ],
    compiler_params=pltpu.CompilerParams(
        needs_layout_passes=False, use_tc_tiling_on_sc=True
    ),
)
def _mf_gather(user_hbm, item_hbm, uwt_hbm, iwt_hbm, tailu_hbm, taili_hbm,
               xu_hbm, xi_hbm,
               idx_all, pb, selb, stage3, rowst3, sidx2,
               sem_st, sem_s0, sem_s1):
    wid = lax.axis_index("s") * NC + lax.axis_index("c")
    lane = lax.iota(jnp.int32, L)
    count_t = jnp.where(wid >= 28, 245, 244)
    start_t = 244 * wid + jnp.maximum(wid - 28, 0)
    r_slab_lo = start_t * 128
    r_slab_hi = jnp.where(wid == NW - 1, N_ROWS, (start_t + count_t) * 128)
    nchunk = (count_t + 3) >> 2
    nchunk_eff = nchunk + jnp.where(wid == NW - 1, 1, 0)

    def sweep(idx_hbm, tbl_hbm, tail_hbm, x_hbm):
        pltpu.sync_copy(idx_hbm, idx_all)

        def scan_all(q, n):
            bvec = q * L + lane
            rvec = idx_all[pl.ds(q * L, L)]
            m = (rvec >= r_slab_lo) & (rvec < r_slab_hi)
            c = plsc.cumsum(m.astype(jnp.int32))
            plsc.store_scatter(pb, [n + c - 1], bvec, mask=m)
            return n + c[L - 1]

        n_pairs = lax.fori_loop(0, B // L, scan_all, 0)
        nq = (n_pairs + 15) >> 4

        def start_chunk_dma(ci):
            off = (start_t + jnp.minimum(4 * ci, count_t - 4)) * 128
            pltpu.async_copy(
                tbl_hbm.at[:, pl.ds(off, CHUNK)], stage3.at[ci % 2], sem_st
            )

        start_chunk_dma(0)

        def chunk_body(ci, par, sem_sc):
            is_tail = ci == nchunk
            sl = ci % 2
            r_lo = jnp.where(
                is_tail, TAIL_LO,
                (start_t + jnp.minimum(4 * ci, count_t - 4)) * 128,
            )

            @pl.when(ci + 1 < nchunk)
            def _():
                start_chunk_dma(ci + 1)

            @pl.when((ci + 1 == nchunk) & (wid == NW - 1))
            def _():
                pltpu.async_copy(
                    tail_hbm, stage3.at[(ci + 1) % 2, :, pl.ds(0, 128)], sem_st
                )

            @pl.when(jnp.logical_not(is_tail))
            def _():
                pltpu.make_async_copy(
                    tbl_hbm.at[:, pl.ds(0, CHUNK)], stage3.at[sl], sem_st
                ).wait()

            @pl.when(is_tail)
            def _():
                pltpu.make_async_copy(
                    tail_hbm, stage3.at[sl, :, pl.ds(0, 128)], sem_st
                ).wait()

            @pl.when(ci >= 2)
            def _():
                pltpu.make_async_copy(
                    rowst3.at[par], x_hbm.at[plsc.Indices(sidx2.at[par], ignored_value=DUMP)], sem_sc
                ).wait()

            for k in range(4):
                sidx2[par, pl.ds(k * L, L)] = jnp.full((L,), DUMP, jnp.int32)

            def scan_pairs(q, n):
                valid = (q * L + lane) < n_pairs
                bvec = pb[pl.ds(q * L, L)]
                bs = jnp.minimum(jnp.maximum(bvec, 0), B - 1)
                rvec = plsc.load_gather(idx_all, [bs])
                m = valid & (rvec >= r_lo) & (rvec < r_lo + CHUNK)
                c = plsc.cumsum(m.astype(jnp.int32))
                pos = jnp.minimum(n + c - 1, SEL_CAP - 1)
                plsc.store_scatter(selb, [pos], bvec, mask=m)
                return n + c[L - 1]

            n_sel = lax.fori_loop(0, nq, scan_pairs, 0)
            ngroups = (n_sel + 15) >> 4

            par_v = jnp.full((L,), par, jnp.int32)
            sl_v = jnp.broadcast_to(sl, (L,)).astype(jnp.int32)

            def group(g, carry):
                valid = (g * L + lane) < n_sel
                b16 = selb[pl.ds((g % (SEL_CAP // L)) * L, L)]
                b16 = jnp.where(valid, b16, DUMP)
                bs = jnp.minimum(jnp.maximum(b16, 0), B - 1)
                rloc = plsc.load_gather(idx_all, [bs]) - r_lo
                rloc = jnp.minimum(jnp.maximum(rloc, 0), CHUNK - 1)
                slot = g % 4
                row16 = slot * L + lane
                for d in range(D):
                    d_v = jnp.full((L,), d, jnp.int32)
                    vals = plsc.load_gather(stage3, [sl_v, d_v, rloc])
                    plsc.store_scatter(rowst3, [par_v, row16, d_v], vals)
                sidx2[par, pl.ds(slot * L, L)] = b16

                @pl.when((slot == 3) & (g + 1 < ngroups))
                def _():
                    pltpu.async_copy(
                        rowst3.at[par], x_hbm.at[plsc.Indices(sidx2.at[par], ignored_value=DUMP)], sem_sc
                    ).wait()

                return carry

            lax.fori_loop(0, ngroups, group, 0)
            pltpu.async_copy(rowst3.at[par], x_hbm.at[plsc.Indices(sidx2.at[par], ignored_value=DUMP)], sem_sc)

        def body(ci, carry):
            @pl.when(ci % 2 == 0)
            def _():
                chunk_body(ci, 0, sem_s0)

            @pl.when(ci % 2 == 1)
            def _():
                chunk_body(ci, 1, sem_s1)

            return carry

        lax.fori_loop(0, nchunk_eff, body, 0)

        pltpu.make_async_copy(
            rowst3.at[0], x_hbm.at[plsc.Indices(sidx2.at[0], ignored_value=DUMP)], sem_s0
        ).wait()
        pltpu.make_async_copy(
            rowst3.at[1], x_hbm.at[plsc.Indices(sidx2.at[1], ignored_value=DUMP)], sem_s1
        ).wait()

    sweep(user_hbm, uwt_hbm, tailu_hbm, xu_hbm)
    sweep(item_hbm, iwt_hbm, taili_hbm, xi_hbm)


def _dot_body(xu_ref, xi_ref, o_ref):
    u = xu_ref[:, :D]
    it = xi_ref[:, :D]
    o_ref[...] = jnp.sum(u * it, axis=1)


_dot = pl.pallas_call(
    _dot_body,
    grid=(16,),
    in_specs=[
        pl.BlockSpec((1024, 128), lambda i: (i, 0)),
        pl.BlockSpec((1024, 128), lambda i: (i, 0)),
    ],
    out_specs=pl.BlockSpec((1024,), lambda i: (i,)),
    out_shape=jax.ShapeDtypeStruct((B,), jnp.float32),
)


def kernel(user, item, user_weight, item_weight):
    uwt = user_weight.T
    iwt = item_weight.T
    tail_u = jnp.pad(uwt[:, TAIL_LO:], ((0, 0), (0, 64)))
    tail_i = jnp.pad(iwt[:, TAIL_LO:], ((0, 0), (0, 64)))
    xu, xi = _mf_gather(user, item, uwt, iwt, tail_u, tail_i)
    return _dot(xu[:B], xi[:B])


# speedup vs baseline: 1.6522x; 1.1320x over previous
import functools

import jax
import jax.numpy as jnp
from jax import lax
from jax.experimental import pallas as pl
from jax.experimental.pallas import tpu as pltpu
from jax.experimental.pallas import tpu_sc as plsc

NC, NS, L = 2, 16, 16
NW = NC * NS
B = 16384
D = 64
N_ROWS = 1000000
FULL_TILES = N_ROWS // 128
TAIL_LO = FULL_TILES * 128
CHUNK = 512
XROWS = B + 8
SEL_CAP = 2048
DUMP = B

_mesh = plsc.VectorSubcoreMesh(core_axis_name="c", subcore_axis_name="s")


@functools.partial(
    pl.kernel,
    out_type=(
        jax.ShapeDtypeStruct((XROWS, 128), jnp.float32),
        jax.ShapeDtypeStruct((XROWS, 128), jnp.float32),
    ),
    mesh=_mesh,
    scratch_types=[
        pltpu.VMEM((B,), jnp.int32),
        pltpu.VMEM((B,), jnp.int32),
        pltpu.VMEM((SEL_CAP,), jnp.int32),
        pltpu.VMEM((2, D, CHUNK), jnp.float32),
        pltpu.VMEM((2, 64, 128), jnp.float32),
        pltpu.VMEM((2, 64), jnp.int32),
        pltpu.SemaphoreType.DMA,
        pltpu.SemaphoreType.DMA,
        pltpu.SemaphoreType.DMA,
    ],
    compiler_params=pltpu.CompilerParams(
        needs_layout_passes=False, use_tc_tiling_on_sc=True
    ),
)
def _mf_gather(user_hbm, item_hbm, uwt_hbm, iwt_hbm, tailu_hbm, taili_hbm,
               xu_hbm, xi_hbm,
               idx_all, pb, selb, stage3, rowst3, sidx2,
               sem_st, sem_s0, sem_s1):
    wid = lax.axis_index("s") * NC + lax.axis_index("c")
    lane = lax.iota(jnp.int32, L)
    count_t = jnp.where(wid >= 28, 245, 244)
    start_t = 244 * wid + jnp.maximum(wid - 28, 0)
    r_slab_lo = start_t * 128
    r_slab_hi = jnp.where(wid == NW - 1, N_ROWS, (start_t + count_t) * 128)
    nchunk = (count_t + 3) >> 2
    nchunk_eff = nchunk + jnp.where(wid == NW - 1, 1, 0)

    def sweep(idx_hbm, tbl_hbm, tail_hbm, x_hbm):
        pltpu.sync_copy(idx_hbm, idx_all)

        def scan_all(q, n):
            bvec = q * L + lane
            rvec = idx_all[pl.ds(q * L, L)]
            m = (rvec >= r_slab_lo) & (rvec < r_slab_hi)
            c = plsc.cumsum(m.astype(jnp.int32))
            plsc.store_scatter(pb, [n + c - 1], bvec, mask=m)
            return n + c[L - 1]

        n_pairs = lax.fori_loop(0, B // L, scan_all, 0)
        nq = (n_pairs + 15) >> 4

        def start_chunk_dma(ci):
            off = (start_t + jnp.minimum(4 * ci, count_t - 4)) * 128
            pltpu.async_copy(
                tbl_hbm.at[:, pl.ds(off, CHUNK)], stage3.at[ci % 2], sem_st
            )

        start_chunk_dma(0)

        def chunk_body(ci, par, sem_sc):
            is_tail = ci == nchunk
            sl = ci % 2
            r_lo = jnp.where(
                is_tail, TAIL_LO,
                (start_t + jnp.minimum(4 * ci, count_t - 4)) * 128,
            )

            @pl.when(ci + 1 < nchunk)
            def _():
                start_chunk_dma(ci + 1)

            @pl.when((ci + 1 == nchunk) & (wid == NW - 1))
            def _():
                pltpu.async_copy(
                    tail_hbm, stage3.at[(ci + 1) % 2, :, pl.ds(0, 128)], sem_st
                )

            @pl.when(ci >= 2)
            def _():
                pltpu.make_async_copy(
                    rowst3.at[par], x_hbm.at[plsc.Indices(sidx2.at[par], ignored_value=DUMP)], sem_sc
                ).wait()

            for k in range(4):
                sidx2[par, pl.ds(k * L, L)] = jnp.full((L,), DUMP, jnp.int32)

            def scan_pairs(q, n):
                valid = (q * L + lane) < n_pairs
                bvec = pb[pl.ds(q * L, L)]
                bs = jnp.minimum(jnp.maximum(bvec, 0), B - 1)
                rvec = plsc.load_gather(idx_all, [bs])
                m = valid & (rvec >= r_lo) & (rvec < r_lo + CHUNK)
                c = plsc.cumsum(m.astype(jnp.int32))
                pos = jnp.minimum(n + c - 1, SEL_CAP - 1)
                plsc.store_scatter(selb, [pos], bvec, mask=m)
                return n + c[L - 1]

            n_sel = lax.fori_loop(0, nq, scan_pairs, 0)
            ngroups = (n_sel + 15) >> 4

            @pl.when(jnp.logical_not(is_tail))
            def _():
                pltpu.make_async_copy(
                    tbl_hbm.at[:, pl.ds(0, CHUNK)], stage3.at[sl], sem_st
                ).wait()

            @pl.when(is_tail)
            def _():
                pltpu.make_async_copy(
                    tail_hbm, stage3.at[sl, :, pl.ds(0, 128)], sem_st
                ).wait()


            par_v = jnp.full((L,), par, jnp.int32)
            sl_v = jnp.broadcast_to(sl, (L,)).astype(jnp.int32)

            def group(g, carry):
                valid = (g * L + lane) < n_sel
                b16 = selb[pl.ds((g % (SEL_CAP // L)) * L, L)]
                b16 = jnp.where(valid, b16, DUMP)
                bs = jnp.minimum(jnp.maximum(b16, 0), B - 1)
                rloc = plsc.load_gather(idx_all, [bs]) - r_lo
                rloc = jnp.minimum(jnp.maximum(rloc, 0), CHUNK - 1)
                slot = g % 4
                row16 = slot * L + lane
                for d in range(D):
                    d_v = jnp.full((L,), d, jnp.int32)
                    vals = plsc.load_gather(stage3, [sl_v, d_v, rloc])
                    plsc.store_scatter(rowst3, [par_v, row16, d_v], vals)
                sidx2[par, pl.ds(slot * L, L)] = b16

                @pl.when((slot == 3) & (g + 1 < ngroups))
                def _():
                    pltpu.async_copy(
                        rowst3.at[par], x_hbm.at[plsc.Indices(sidx2.at[par], ignored_value=DUMP)], sem_sc
                    ).wait()

                return carry

            lax.fori_loop(0, ngroups, group, 0)
            pltpu.async_copy(rowst3.at[par], x_hbm.at[plsc.Indices(sidx2.at[par], ignored_value=DUMP)], sem_sc)

        def body(ci, carry):
            @pl.when(ci % 2 == 0)
            def _():
                chunk_body(ci, 0, sem_s0)

            @pl.when(ci % 2 == 1)
            def _():
                chunk_body(ci, 1, sem_s1)

            return carry

        lax.fori_loop(0, nchunk_eff, body, 0)

        pltpu.make_async_copy(
            rowst3.at[0], x_hbm.at[plsc.Indices(sidx2.at[0], ignored_value=DUMP)], sem_s0
        ).wait()
        pltpu.make_async_copy(
            rowst3.at[1], x_hbm.at[plsc.Indices(sidx2.at[1], ignored_value=DUMP)], sem_s1
        ).wait()

    sweep(user_hbm, uwt_hbm, tailu_hbm, xu_hbm)
    sweep(item_hbm, iwt_hbm, taili_hbm, xi_hbm)


def _dot_body(xu_ref, xi_ref, o_ref):
    u = xu_ref[:, :D]
    it = xi_ref[:, :D]
    o_ref[...] = jnp.sum(u * it, axis=1)


_dot = pl.pallas_call(
    _dot_body,
    grid=(16,),
    in_specs=[
        pl.BlockSpec((1024, 128), lambda i: (i, 0)),
        pl.BlockSpec((1024, 128), lambda i: (i, 0)),
    ],
    out_specs=pl.BlockSpec((1024,), lambda i: (i,)),
    out_shape=jax.ShapeDtypeStruct((B,), jnp.float32),
)


def kernel(user, item, user_weight, item_weight):
    uwt = user_weight.T
    iwt = item_weight.T
    tail_u = jnp.pad(uwt[:, TAIL_LO:], ((0, 0), (0, 64)))
    tail_i = jnp.pad(iwt[:, TAIL_LO:], ((0, 0), (0, 64)))
    xu, xi = _mf_gather(user, item, uwt, iwt, tail_u, tail_i)
    return _dot(xu[:B], xi[:B])


# speedup vs baseline: 1.6667x; 1.0088x over previous
import functools

import jax
import jax.numpy as jnp
from jax import lax
from jax.experimental import pallas as pl
from jax.experimental.pallas import tpu as pltpu
from jax.experimental.pallas import tpu_sc as plsc

NC, NS, L = 2, 16, 16
NW = NC * NS
B = 16384
D = 64
N_ROWS = 1000000
FULL_TILES = N_ROWS // 128
TAIL_LO = FULL_TILES * 128
CHUNK = 512
XROWS = B + 8
SEL_CAP = 2048
DUMP = B

_mesh = plsc.VectorSubcoreMesh(core_axis_name="c", subcore_axis_name="s")


@functools.partial(
    pl.kernel,
    out_type=(
        jax.ShapeDtypeStruct((XROWS, 128), jnp.float32),
        jax.ShapeDtypeStruct((XROWS, 128), jnp.float32),
    ),
    mesh=_mesh,
    scratch_types=[
        pltpu.VMEM((B,), jnp.int32),
        pltpu.VMEM((B,), jnp.int32),
        pltpu.VMEM((SEL_CAP,), jnp.int32),
        pltpu.VMEM((2, D, CHUNK), jnp.float32),
        pltpu.VMEM((2, 64, 128), jnp.float32),
        pltpu.VMEM((2, 64), jnp.int32),
        pltpu.SemaphoreType.DMA,
        pltpu.SemaphoreType.DMA,
        pltpu.SemaphoreType.DMA,
    ],
    compiler_params=pltpu.CompilerParams(
        needs_layout_passes=False, use_tc_tiling_on_sc=True
    ),
)
def _mf_gather(user_hbm, item_hbm, uwt_hbm, iwt_hbm, tailu_hbm, taili_hbm,
               xu_hbm, xi_hbm,
               idx_all, pb, selb, stage3, rowst3, sidx2,
               sem_st, sem_s0, sem_s1):
    wid = lax.axis_index("s") * NC + lax.axis_index("c")
    lane = lax.iota(jnp.int32, L)
    count_t = jnp.where(wid >= 28, 245, 244)
    start_t = 244 * wid + jnp.maximum(wid - 28, 0)
    r_slab_lo = start_t * 128
    r_slab_hi = jnp.where(wid == NW - 1, N_ROWS, (start_t + count_t) * 128)
    nchunk = (count_t + 3) >> 2
    nchunk_eff = nchunk + jnp.where(wid == NW - 1, 1, 0)

    def sweep(idx_hbm, tbl_hbm, tail_hbm, x_hbm):
        pltpu.sync_copy(idx_hbm, idx_all)

        def start_chunk_dma(ci):
            off = (start_t + jnp.minimum(4 * ci, count_t - 4)) * 128
            pltpu.async_copy(
                tbl_hbm.at[:, pl.ds(off, CHUNK)], stage3.at[ci % 2], sem_st
            )

        start_chunk_dma(0)

        def scan_all(q, n):
            bvec = q * L + lane
            rvec = idx_all[pl.ds(q * L, L)]
            m = (rvec >= r_slab_lo) & (rvec < r_slab_hi)
            c = plsc.cumsum(m.astype(jnp.int32))
            plsc.store_scatter(pb, [n + c - 1], bvec, mask=m)
            return n + c[L - 1]

        n_pairs = lax.fori_loop(0, B // L, scan_all, 0)
        nq = (n_pairs + 15) >> 4

        def chunk_body(ci, par, sem_sc):
            is_tail = ci == nchunk
            sl = ci % 2
            r_lo = jnp.where(
                is_tail, TAIL_LO,
                (start_t + jnp.minimum(4 * ci, count_t - 4)) * 128,
            )

            @pl.when(ci + 1 < nchunk)
            def _():
                start_chunk_dma(ci + 1)

            @pl.when((ci + 1 == nchunk) & (wid == NW - 1))
            def _():
                pltpu.async_copy(
                    tail_hbm, stage3.at[(ci + 1) % 2, :, pl.ds(0, 128)], sem_st
                )

            @pl.when(ci >= 2)
            def _():
                pltpu.make_async_copy(
                    rowst3.at[par], x_hbm.at[plsc.Indices(sidx2.at[par], ignored_value=DUMP)], sem_sc
                ).wait()

            for k in range(4):
                sidx2[par, pl.ds(k * L, L)] = jnp.full((L,), DUMP, jnp.int32)

            def scan_pairs(q, n):
                valid = (q * L + lane) < n_pairs
                bvec = pb[pl.ds(q * L, L)]
                bs = jnp.minimum(jnp.maximum(bvec, 0), B - 1)
                rvec = plsc.load_gather(idx_all, [bs])
                m = valid & (rvec >= r_lo) & (rvec < r_lo + CHUNK)
                c = plsc.cumsum(m.astype(jnp.int32))
                pos = jnp.minimum(n + c - 1, SEL_CAP - 1)
                plsc.store_scatter(selb, [pos], bvec, mask=m)
                return n + c[L - 1]

            n_sel = lax.fori_loop(0, nq, scan_pairs, 0)
            ngroups = (n_sel + 15) >> 4

            @pl.when(jnp.logical_not(is_tail))
            def _():
                pltpu.make_async_copy(
                    tbl_hbm.at[:, pl.ds(0, CHUNK)], stage3.at[sl], sem_st
                ).wait()

            @pl.when(is_tail)
            def _():
                pltpu.make_async_copy(
                    tail_hbm, stage3.at[sl, :, pl.ds(0, 128)], sem_st
                ).wait()


            par_v = jnp.full((L,), par, jnp.int32)
            sl_v = jnp.broadcast_to(sl, (L,)).astype(jnp.int32)

            def group(g, carry):
                valid = (g * L + lane) < n_sel
                b16 = selb[pl.ds((g % (SEL_CAP // L)) * L, L)]
                b16 = jnp.where(valid, b16, DUMP)
                bs = jnp.minimum(jnp.maximum(b16, 0), B - 1)
                rloc = plsc.load_gather(idx_all, [bs]) - r_lo
                rloc = jnp.minimum(jnp.maximum(rloc, 0), CHUNK - 1)
                slot = g % 4
                row16 = slot * L + lane
                for d in range(D):
                    d_v = jnp.full((L,), d, jnp.int32)
                    vals = plsc.load_gather(stage3, [sl_v, d_v, rloc])
                    plsc.store_scatter(rowst3, [par_v, row16, d_v], vals)
                sidx2[par, pl.ds(slot * L, L)] = b16

                @pl.when((slot == 3) & (g + 1 < ngroups))
                def _():
                    pltpu.async_copy(
                        rowst3.at[par], x_hbm.at[plsc.Indices(sidx2.at[par], ignored_value=DUMP)], sem_sc
                    ).wait()

                return carry

            lax.fori_loop(0, ngroups, group, 0)
            pltpu.async_copy(rowst3.at[par], x_hbm.at[plsc.Indices(sidx2.at[par], ignored_value=DUMP)], sem_sc)

        def body(ci, carry):
            @pl.when(ci % 2 == 0)
            def _():
                chunk_body(ci, 0, sem_s0)

            @pl.when(ci % 2 == 1)
            def _():
                chunk_body(ci, 1, sem_s1)

            return carry

        lax.fori_loop(0, nchunk_eff, body, 0)

        pltpu.make_async_copy(
            rowst3.at[0], x_hbm.at[plsc.Indices(sidx2.at[0], ignored_value=DUMP)], sem_s0
        ).wait()
        pltpu.make_async_copy(
            rowst3.at[1], x_hbm.at[plsc.Indices(sidx2.at[1], ignored_value=DUMP)], sem_s1
        ).wait()

    sweep(user_hbm, uwt_hbm, tailu_hbm, xu_hbm)
    sweep(item_hbm, iwt_hbm, taili_hbm, xi_hbm)


def _dot_body(xu_ref, xi_ref, o_ref):
    u = xu_ref[:, :D]
    it = xi_ref[:, :D]
    o_ref[...] = jnp.sum(u * it, axis=1)


_dot = pl.pallas_call(
    _dot_body,
    grid=(16,),
    in_specs=[
        pl.BlockSpec((1024, 128), lambda i: (i, 0)),
        pl.BlockSpec((1024, 128), lambda i: (i, 0)),
    ],
    out_specs=pl.BlockSpec((1024,), lambda i: (i,)),
    out_shape=jax.ShapeDtypeStruct((B,), jnp.float32),
)


def kernel(user, item, user_weight, item_weight):
    uwt = user_weight.T
    iwt = item_weight.T
    tail_u = jnp.pad(uwt[:, TAIL_LO:], ((0, 0), (0, 64)))
    tail_i = jnp.pad(iwt[:, TAIL_LO:], ((0, 0), (0, 64)))
    xu, xi = _mf_gather(user, item, uwt, iwt, tail_u, tail_i)
    return _dot(xu[:B], xi[:B])
